# baseline (speedup 1.0000x reference)
import functools

import jax
import jax.numpy as jnp
from jax import lax
from jax.experimental import pallas as pl
from jax.experimental.pallas import tpu as pltpu
from jax.experimental.pallas import tpu_sc as plsc

_VOCAB = 100000
_DIM = 128
_B = 4096
_NEG = 20
_NW = 32
_IPW = _B // _NW
_CI = 8
_CR = _CI * _NEG
_NCH = _IPW // _CI
_LANES = 16
_NSW = _IPW * _NEG


def _sc_scores(target_table, context_table, target_idx, context_idx, neg_idx):
    mesh = plsc.VectorSubcoreMesh(core_axis_name="c", subcore_axis_name="s")

    @functools.partial(
        pl.kernel,
        mesh=mesh,
        out_type=(jax.ShapeDtypeStruct((_B * _NEG,), jnp.float32),
                  jax.ShapeDtypeStruct((_B,), jnp.float32)),
        compiler_params=pltpu.CompilerParams(needs_layout_passes=False),
        scratch_types=[
            pltpu.VMEM((_IPW,), jnp.int32),
            pltpu.VMEM((_IPW,), jnp.int32),
            pltpu.VMEM((_IPW * _NEG,), jnp.int32),
            pltpu.VMEM((_IPW, _DIM), jnp.float32),
            pltpu.VMEM((_IPW, _DIM), jnp.float32),
            pltpu.VMEM((_CR, _DIM), jnp.float32),
            pltpu.VMEM((_CR, _DIM), jnp.float32),
            pltpu.VMEM((_NSW,), jnp.float32),
            pltpu.VMEM((_IPW,), jnp.float32),
            pltpu.SemaphoreType.DMA,
            pltpu.SemaphoreType.DMA,
            pltpu.SemaphoreType.DMA,
            pltpu.SemaphoreType.DMA,
        ],
    )
    def k(ttab, ctab, tidx, cidx, nidx, outn, outp, tixv, cixv, nixv,
          trows, cprows, nrows_a, nrows_b, nsc, psc,
          sem_t, sem_p, sem_a, sem_b):
        wid = lax.axis_index("s") * 2 + lax.axis_index("c")
        ibase = wid * _IPW
        nbase = ibase * _NEG
        pltpu.sync_copy(nidx.at[pl.ds(nbase, _IPW * _NEG)], nixv)
        pltpu.sync_copy(tidx.at[pl.ds(ibase, _IPW)], tixv)
        pltpu.sync_copy(cidx.at[pl.ds(ibase, _IPW)], cixv)
        tcopy = pltpu.async_copy(ttab.at[tixv], trows, sem_t)
        pcopy = pltpu.async_copy(ctab.at[cixv], cprows, sem_p)

        nrows = (nrows_a, nrows_b)
        gsem = (sem_a, sem_b)

        lane = lax.iota(jnp.int32, _LANES)
        m0 = lane == 0
        perms = [lane ^ 8, lane ^ 4, lane ^ 2, lane ^ 1]
        p8, p4, p2, p1 = perms
        lt8 = lane < 8
        l4 = (lane & 4) == 0
        mq = (lane & 3) == 0
        offv = ((lane >> 2) & 1) * 2 + ((lane >> 3) & 1)

        def hsum(acc):
            for p in perms:
                acc = acc + acc[p]
            return acc

        def quad(a, b, c, d):
            a2 = a + a[p8]
            b2 = b + b[p8]
            c2 = c + c[p8]
            d2 = d + d[p8]
            mab = jnp.where(lt8, a2, b2)
            mcd = jnp.where(lt8, c2, d2)
            mab = mab + mab[p4]
            mcd = mcd + mcd[p4]
            m = jnp.where(l4, mab, mcd[p4])
            m = m + m[p2]
            return m + m[p1]

        def issue(ch):
            b = ch & 1
            r0 = ch * _CR
            c1 = pltpu.async_copy(
                ctab.at[nixv.at[pl.ds(r0, 96)]], nrows[b].at[pl.ds(0, 96)],
                gsem[b])
            c2 = pltpu.async_copy(
                ctab.at[nixv.at[pl.ds(r0 + 96, 64)]],
                nrows[b].at[pl.ds(96, 64)], gsem[b])
            return (c1, c2)

        pending = issue(0)
        tcopy.wait()
        pcopy.wait()

        @plsc.parallel_loop(0, _IPW, unroll=4)
        def _(i):
            acc = (trows[i, pl.ds(0, _LANES)] * cprows[i, pl.ds(0, _LANES)])
            for c in range(1, 8):
                acc = acc + (trows[i, pl.ds(c * _LANES, _LANES)]
                             * cprows[i, pl.ds(c * _LANES, _LANES)])
            plsc.store_scatter(psc, [jnp.full((_LANES,), i, jnp.int32)],
                               hsum(acc), mask=m0)

        for ch in range(_NCH):
            b = ch & 1
            nxt = issue(ch + 1) if ch + 1 < _NCH else None
            pending[0].wait()
            pending[1].wait()
            pending = nxt
            nb = nrows[b]

            def item_body(i, carry, _nb=nb, _ch=ch):
                item = _ch * _CI + i
                tv = [trows[item, pl.ds(c * _LANES, _LANES)]
                      for c in range(8)]

                @plsc.parallel_loop(0, _NEG // 4, unroll=5)
                def _(g):
                    f0 = i * _NEG + g * 4
                    accs = []
                    for q in range(4):
                        f = f0 + q
                        acc = tv[0] * _nb[f, pl.ds(0, _LANES)]
                        for c in range(1, 8):
                            acc = acc + (tv[c]
                                         * _nb[f, pl.ds(c * _LANES, _LANES)])
                        accs.append(acc)
                    idxv = jnp.full((_LANES,), _ch * _CR + f0, jnp.int32) + offv
                    plsc.store_scatter(nsc, [idxv], quad(*accs), mask=mq)

                return carry

            lax.fori_loop(0, _CI, item_body, 0)

        pltpu.sync_copy(nsc, outn.at[pl.ds(nbase, _NSW)])
        pltpu.sync_copy(psc, outp.at[pl.ds(ibase, _IPW)])

    return k(target_table, context_table, target_idx, context_idx, neg_idx)


def _tc_loss(neg_s, pos_s):
    nrow = _B * _NEG // 128
    prow = _B // 128

    def body(xn_ref, xp_ref, o_ref):
        xn = xn_ref[...]
        xp = xp_ref[...]
        lin = (jnp.sum(jnp.minimum(xp, 0.0))
               + jnp.sum(jnp.minimum(-xn, 0.0)))
        pn = 1.0 + jnp.exp(-jnp.abs(xn))
        pp = 1.0 + jnp.exp(-jnp.abs(xp))
        n = nrow
        while n > 40:
            n //= 2
            pn = pn[:n, :] * pn[n:2 * n, :]
        m = prow
        while m > 8:
            m //= 2
            pp = pp[:m, :] * pp[m:2 * m, :]
        logs = jnp.sum(jnp.log(pn)) + jnp.sum(jnp.log(pp))
        o_ref[...] = (-(lin - logs) / _B).reshape(1, 1)

    return pl.pallas_call(
        body,
        out_shape=jax.ShapeDtypeStruct((1, 1), jnp.float32),
    )(neg_s.reshape(nrow, 128), pos_s.reshape(prow, 128))


def kernel(target_table, context_table, target_idx, context_idx, neg_idx):
    tidx = target_idx.astype(jnp.int32)
    cidx = context_idx.astype(jnp.int32)
    nidx = neg_idx.astype(jnp.int32).reshape(-1)
    neg_s, pos_s = _sc_scores(target_table, context_table, tidx, cidx, nidx)
    loss = _tc_loss(neg_s, pos_s)
    return loss[0, 0]


# speedup vs baseline: 1.0512x; 1.0512x over previous
import functools

import jax
import jax.numpy as jnp
from jax import lax
from jax.experimental import pallas as pl
from jax.experimental.pallas import tpu as pltpu
from jax.experimental.pallas import tpu_sc as plsc

_VOCAB = 100000
_DIM = 128
_B = 4096
_NEG = 20
_NW = 32
_IPW = _B // _NW
_CI = 8
_CR = _CI * _NEG
_NCH = _IPW // _CI
_LANES = 16
_NSW = _IPW * _NEG


def _sc_scores(target_table, context_table, target_idx, context_idx, neg_idx):
    mesh = plsc.VectorSubcoreMesh(core_axis_name="c", subcore_axis_name="s")

    @functools.partial(
        pl.kernel,
        mesh=mesh,
        out_type=(jax.ShapeDtypeStruct((_B * _NEG,), jnp.float32),
                  jax.ShapeDtypeStruct((_B,), jnp.float32)),
        compiler_params=pltpu.CompilerParams(needs_layout_passes=False),
        scratch_types=[
            pltpu.VMEM((_IPW,), jnp.int32),
            pltpu.VMEM((_IPW,), jnp.int32),
            pltpu.VMEM((_IPW * _NEG,), jnp.int32),
            pltpu.VMEM((_IPW, _DIM), jnp.float32),
            pltpu.VMEM((_IPW, _DIM), jnp.float32),
            pltpu.VMEM((_CR, _DIM), jnp.float32),
            pltpu.VMEM((_CR, _DIM), jnp.float32),
            pltpu.VMEM((_NSW,), jnp.float32),
            pltpu.VMEM((_IPW,), jnp.float32),
            pltpu.SemaphoreType.DMA,
            pltpu.SemaphoreType.DMA,
            pltpu.SemaphoreType.DMA,
            pltpu.SemaphoreType.DMA,
            pltpu.SemaphoreType.DMA,
            pltpu.SemaphoreType.DMA,
        ],
    )
    def k(ttab, ctab, tidx, cidx, nidx, outn, outp, tixv, cixv, nixv,
          trows, cprows, nrows_a, nrows_b, nsc, psc,
          sem_t, sem_p, sem_a, sem_b, sem_ni, sem_tc):
        wid = lax.axis_index("s") * 2 + lax.axis_index("c")
        ibase = wid * _IPW
        nbase = ibase * _NEG
        nicopy = pltpu.async_copy(nidx.at[pl.ds(nbase, _IPW * _NEG)], nixv,
                                  sem_ni)
        ticopy = pltpu.async_copy(tidx.at[pl.ds(ibase, _IPW)], tixv, sem_tc)
        cicopy = pltpu.async_copy(cidx.at[pl.ds(ibase, _IPW)], cixv, sem_tc)

        nrows = (nrows_a, nrows_b)
        gsem = (sem_a, sem_b)

        lane = lax.iota(jnp.int32, _LANES)
        m0 = lane == 0
        perms = [lane ^ 8, lane ^ 4, lane ^ 2, lane ^ 1]

        def hsum(acc):
            for p in perms:
                acc = acc + acc[p]
            return acc

        def issue(ch):
            b = ch & 1
            r0 = ch * _CR
            c1 = pltpu.async_copy(
                ctab.at[nixv.at[pl.ds(r0, 96)]], nrows[b].at[pl.ds(0, 96)],
                gsem[b])
            c2 = pltpu.async_copy(
                ctab.at[nixv.at[pl.ds(r0 + 96, 64)]],
                nrows[b].at[pl.ds(96, 64)], gsem[b])
            return (c1, c2)

        nicopy.wait()
        pending = issue(0)
        ticopy.wait()
        cicopy.wait()
        tcopy = pltpu.async_copy(ttab.at[tixv], trows, sem_t)
        pcopy = pltpu.async_copy(ctab.at[cixv], cprows, sem_p)
        tcopy.wait()
        pcopy.wait()

        @plsc.parallel_loop(0, _IPW, unroll=4)
        def _(i):
            acc = (trows[i, pl.ds(0, _LANES)] * cprows[i, pl.ds(0, _LANES)])
            for c in range(1, 8):
                acc = acc + (trows[i, pl.ds(c * _LANES, _LANES)]
                             * cprows[i, pl.ds(c * _LANES, _LANES)])
            plsc.store_scatter(psc, [jnp.full((_LANES,), i, jnp.int32)],
                               hsum(acc), mask=m0)

        for ch in range(_NCH):
            b = ch & 1
            nxt = issue(ch + 1) if ch + 1 < _NCH else None
            pending[0].wait()
            pending[1].wait()
            pending = nxt
            nb = nrows[b]

            def item_body(i, carry, _nb=nb, _ch=ch):
                item = _ch * _CI + i
                tv = [trows[item, pl.ds(c * _LANES, _LANES)]
                      for c in range(8)]

                @plsc.parallel_loop(0, _NEG, unroll=5)
                def _(kk):
                    f = i * _NEG + kk
                    acc = tv[0] * _nb[f, pl.ds(0, _LANES)]
                    for c in range(1, 8):
                        acc = acc + tv[c] * _nb[f, pl.ds(c * _LANES, _LANES)]
                    plsc.store_scatter(
                        nsc, [jnp.full((_LANES,), _ch * _CR + f, jnp.int32)],
                        hsum(acc), mask=m0)

                return carry

            lax.fori_loop(0, _CI, item_body, 0)

        pltpu.sync_copy(nsc, outn.at[pl.ds(nbase, _NSW)])
        pltpu.sync_copy(psc, outp.at[pl.ds(ibase, _IPW)])

    return k(target_table, context_table, target_idx, context_idx, neg_idx)


def _tc_loss(neg_s, pos_s):
    nrow = _B * _NEG // 128
    prow = _B // 128

    def body(xn_ref, xp_ref, o_ref):
        xn = xn_ref[...]
        xp = xp_ref[...]
        lin = (jnp.sum(jnp.minimum(xp, 0.0))
               + jnp.sum(jnp.minimum(-xn, 0.0)))
        pn = 1.0 + jnp.exp(-jnp.abs(xn))
        pp = 1.0 + jnp.exp(-jnp.abs(xp))
        n = nrow
        while n > 40:
            n //= 2
            pn = pn[:n, :] * pn[n:2 * n, :]
        m = prow
        while m > 8:
            m //= 2
            pp = pp[:m, :] * pp[m:2 * m, :]
        logs = jnp.sum(jnp.log(pn)) + jnp.sum(jnp.log(pp))
        o_ref[...] = (-(lin - logs) / _B).reshape(1, 1)

    return pl.pallas_call(
        body,
        out_shape=jax.ShapeDtypeStruct((1, 1), jnp.float32),
    )(neg_s.reshape(nrow, 128), pos_s.reshape(prow, 128))


def kernel(target_table, context_table, target_idx, context_idx, neg_idx):
    tidx = target_idx.astype(jnp.int32)
    cidx = context_idx.astype(jnp.int32)
    nidx = neg_idx.astype(jnp.int32).reshape(-1)
    neg_s, pos_s = _sc_scores(target_table, context_table, tidx, cidx, nidx)
    loss = _tc_loss(neg_s, pos_s)
    return loss[0, 0]
